# Initial kernel scaffold; baseline (speedup 1.0000x reference)
import functools

import jax
import jax.numpy as jnp
from jax import lax
from jax.experimental import pallas as pl
from jax.experimental.pallas import tpu as pltpu
from jax.experimental.pallas import tpu_sc as plsc

DIM = 768
HEADS = 8
NUM_KEYS = 256
DIM_KEY = DIM // 2
PK = 8
K = 8
N = 2048
TB = 256
NEG = -1e30


def _topk8(vals, iota, bound):
    scores, idxs, onehots = [], [], []
    v = vals
    for _ in range(8):
        m = jnp.max(v, axis=1, keepdims=True)
        am = jnp.min(jnp.where(v == m, iota, bound), axis=1, keepdims=True)
        sel = iota == am
        scores.append(m)
        idxs.append(am)
        onehots.append(sel)
        v = jnp.where(sel, NEG, v)
    return (jnp.concatenate(scores, axis=1),
            jnp.concatenate(idxs, axis=1).astype(jnp.int32), onehots)


def _phase_a_body(x_ref, g_ref, wq_ref, km_ref, xn_ref, w_ref, idx_ref):
    x = x_ref[...]
    gamma = g_ref[...]
    nrm = jnp.sqrt(jnp.sum(x * x, axis=1, keepdims=True))
    nrm = jnp.maximum(nrm, 1e-12)
    xn = x / nrm * (DIM ** 0.5) * (gamma + 1.0)
    xn_ref[...] = xn

    q = jnp.dot(xn, wq_ref[...], preferred_element_type=jnp.float32)

    iota_k = lax.broadcasted_iota(jnp.int32, (TB, NUM_KEYS), 1)
    iota_64 = lax.broadcasted_iota(jnp.int32, (TB, PK * PK), 1)

    w_parts, idx_parts = [], []
    for h in range(HEADS):
        res = []
        for p in range(2):
            qs = q[:, p * (HEADS * DIM_KEY) + h * DIM_KEY:
                   p * (HEADS * DIM_KEY) + (h + 1) * DIM_KEY]
            sim = jnp.dot(qs, km_ref[p, h], preferred_element_type=jnp.float32)
            res.append(_topk8(sim, iota_k, NUM_KEYS))
        (sx, ix, _), (sy, iy, _) = res

        s_chunks, i_chunks = [], []
        for i in range(PK):
            s_chunks.append(sx[:, i:i + 1] + sy)
            i_chunks.append(ix[:, i:i + 1] * NUM_KEYS + iy)
        all_s = jnp.concatenate(s_chunks, axis=1)
        all_i = jnp.concatenate(i_chunks, axis=1)

        scores_h, _, onehots = _topk8(all_s, iota_64, PK * PK)
        idx_h = []
        for sel in onehots:
            idx_h.append(jnp.sum(jnp.where(sel, all_i, 0), axis=1, keepdims=True))
        idx_h = jnp.concatenate(idx_h, axis=1)

        mx = jnp.max(scores_h, axis=1, keepdims=True)
        e = jnp.exp(scores_h - mx)
        w_parts.append(e / jnp.sum(e, axis=1, keepdims=True))
        idx_parts.append(idx_h)

    w_ref[...] = jnp.concatenate(w_parts, axis=1)
    idx_ref[...] = jnp.concatenate(idx_parts, axis=1)


def _phase_a(x2d, gamma, Wq, Kmat, interpret=False):
    grid = (N // TB,)
    return pl.pallas_call(
        _phase_a_body,
        grid=grid,
        in_specs=[
            pl.BlockSpec((TB, DIM), lambda i: (i, 0)),
            pl.BlockSpec((1, DIM), lambda i: (0, 0)),
            pl.BlockSpec((DIM, 2 * HEADS * DIM_KEY), lambda i: (0, 0)),
            pl.BlockSpec((2, HEADS, DIM_KEY, NUM_KEYS), lambda i: (0, 0, 0, 0)),
        ],
        out_specs=[
            pl.BlockSpec((TB, DIM), lambda i: (i, 0)),
            pl.BlockSpec((TB, HEADS * K), lambda i: (i, 0)),
            pl.BlockSpec((TB, HEADS * K), lambda i: (i, 0)),
        ],
        out_shape=[
            jax.ShapeDtypeStruct((N, DIM), jnp.float32),
            jax.ShapeDtypeStruct((N, HEADS * K), jnp.float32),
            jax.ShapeDtypeStruct((N, HEADS * K), jnp.int32),
        ],
        interpret=interpret,
    )(x2d, gamma.reshape(1, DIM), Wq, Kmat)


def kernel(x, gamma, Wq, keys_p, Wdown, Wup):
    x2d = x.reshape(N, DIM)
    Kmat = keys_p.transpose(2, 0, 3, 1)
    xn, w, idx = _phase_a(x2d, gamma, Wq, Kmat)
    out = _phase_b(xn, w, idx, Wdown, Wup)
    return out.reshape(1, N, DIM)


NC, NS = 2, 16
NW = NC * NS
TPW = N // NW
KH = HEADS * K


def _gelu_w(hvec, wvec):
    z = hvec * 0.7071067811865476
    az = jnp.abs(z)
    t = 1.0 / (1.0 + 0.3275911 * az)
    poly = t * (0.254829592 + t * (-0.284496736 + t * (1.421413741
               + t * (-1.453152027 + t * 1.061405429))))
    erf_abs = 1.0 - poly * jnp.exp(-az * az)
    erf = jnp.where(z < 0, -erf_abs, erf_abs)
    return hvec * 0.5 * (1.0 + erf) * wvec


def _phase_b_body(xn_hbm, w_hbm, idx_hbm, dn_hbm, up_hbm, out_hbm,
                  idx_v, w_v, xn_v, dn_rows, up_rows, g_v, out_v,
                  sem_i, sem_d, sem_u, sem_x, sem_o):
    wid = lax.axis_index("s") * NC + lax.axis_index("c")
    base = wid * TPW

    pltpu.async_copy(idx_hbm.at[pl.ds(base, TPW)], idx_v, sem_i).wait()
    pltpu.async_copy(w_hbm.at[pl.ds(base, TPW)], w_v, sem_i).wait()

    def token(t, carry):
        tok = base + t
        dnc = pltpu.async_copy(dn_hbm.at[idx_v.at[t]], dn_rows, sem_d)
        upc = pltpu.async_copy(up_hbm.at[idx_v.at[t]], up_rows, sem_u)
        pltpu.async_copy(xn_hbm.at[tok], xn_v, sem_x).wait()
        dnc.wait()

        for kg in range(KH // 8):
            def dslice(s, accs, _kg=kg):
                xs = xn_v[pl.ds(s * 16, 16)]
                return tuple(
                    accs[j] + xs * dn_rows[_kg * 8 + j, pl.ds(s * 16, 16)]
                    for j in range(8))
            accs = lax.fori_loop(0, DIM // 16, dslice,
                                 tuple(jnp.zeros((16,), jnp.float32)
                                       for _ in range(8)))
            for j in range(8):
                g_v[pl.ds((kg * 8 + j) * 16, 16)] = accs[j]

        lane = lax.iota(jnp.int32, 16)
        for kg in range(KH // 16):
            tot = jnp.zeros((16,), jnp.float32)
            for c in range(16):
                tot = tot + plsc.load_gather(
                    g_v, [(kg * 16 + lane) * 16 + c])
            wk = w_v[t, pl.ds(kg * 16, 16)]
            g_v[pl.ds(KH * 16 + kg * 16, 16)] = _gelu_w(tot, wk)

        upc.wait()

        for chunk in range(3):
            def kstep(k, accs, _chunk=chunk):
                b = plsc.load_gather(
                    g_v, [jnp.full((16,), KH * 16, jnp.int32) + k])
                return tuple(
                    accs[s] + b * up_rows[k, pl.ds((_chunk * 16 + s) * 16, 16)]
                    for s in range(16))
            accs = lax.fori_loop(0, KH, kstep,
                                 tuple(jnp.zeros((16,), jnp.float32)
                                       for _ in range(16)))
            for s in range(16):
                out_v[pl.ds((chunk * 16 + s) * 16, 16)] = accs[s]

        pltpu.async_copy(out_v, out_hbm.at[tok], sem_o).wait()
        return carry

    lax.fori_loop(0, TPW, token, 0)


def _phase_b(xn, w, idx, Wdown, Wup):
    mesh = plsc.VectorSubcoreMesh(core_axis_name="c", subcore_axis_name="s",
                                  num_cores=NC, num_subcores=NS)
    f = pl.kernel(
        _phase_b_body,
        out_type=jax.ShapeDtypeStruct((N, DIM), jnp.float32),
        mesh=mesh,
        scratch_types=[
            pltpu.VMEM((TPW, KH), jnp.int32),
            pltpu.VMEM((TPW, KH), jnp.float32),
            pltpu.VMEM((DIM,), jnp.float32),
            pltpu.VMEM((KH, DIM), jnp.float32),
            pltpu.VMEM((KH, DIM), jnp.float32),
            pltpu.VMEM((KH * 16 + KH,), jnp.float32),
            pltpu.VMEM((DIM,), jnp.float32),
            pltpu.SemaphoreType.DMA,
            pltpu.SemaphoreType.DMA,
            pltpu.SemaphoreType.DMA,
            pltpu.SemaphoreType.DMA,
            pltpu.SemaphoreType.DMA,
        ],
    )
    return f(xn, w, idx, Wdown, Wup)


# speedup vs baseline: 7.4585x; 7.4585x over previous
import functools

import jax
import jax.numpy as jnp
from jax import lax
from jax.experimental import pallas as pl
from jax.experimental.pallas import tpu as pltpu
from jax.experimental.pallas import tpu_sc as plsc

DIM = 768
HEADS = 8
NUM_KEYS = 256
DIM_KEY = DIM // 2
PK = 8
K = 8
N = 2048
TB = 256
NEG = -1e30


def _topk8(vals, iota, bound):
    scores, idxs, onehots = [], [], []
    v = vals
    for _ in range(8):
        m = jnp.max(v, axis=1, keepdims=True)
        am = jnp.min(jnp.where(v == m, iota, bound), axis=1, keepdims=True)
        sel = iota == am
        scores.append(m)
        idxs.append(am)
        onehots.append(sel)
        v = jnp.where(sel, NEG, v)
    return (jnp.concatenate(scores, axis=1),
            jnp.concatenate(idxs, axis=1).astype(jnp.int32), onehots)


def _phase_a_body(x_ref, g_ref, wq_ref, km_ref, xn_ref, w_ref, idx_ref):
    x = x_ref[...]
    gamma = g_ref[...]
    nrm = jnp.sqrt(jnp.sum(x * x, axis=1, keepdims=True))
    nrm = jnp.maximum(nrm, 1e-12)
    xn = x / nrm * (DIM ** 0.5) * (gamma + 1.0)
    xn_ref[...] = xn

    q = jnp.dot(xn, wq_ref[...], preferred_element_type=jnp.float32)

    iota_k = lax.broadcasted_iota(jnp.int32, (TB, NUM_KEYS), 1)
    iota_64 = lax.broadcasted_iota(jnp.int32, (TB, PK * PK), 1)

    w_parts, idx_parts = [], []
    for h in range(HEADS):
        res = []
        for p in range(2):
            qs = q[:, p * (HEADS * DIM_KEY) + h * DIM_KEY:
                   p * (HEADS * DIM_KEY) + (h + 1) * DIM_KEY]
            sim = jnp.dot(qs, km_ref[p, h], preferred_element_type=jnp.float32)
            res.append(_topk8(sim, iota_k, NUM_KEYS))
        (sx, ix, _), (sy, iy, _) = res

        s_chunks, i_chunks = [], []
        for i in range(PK):
            s_chunks.append(sx[:, i:i + 1] + sy)
            i_chunks.append(ix[:, i:i + 1] * NUM_KEYS + iy)
        all_s = jnp.concatenate(s_chunks, axis=1)
        all_i = jnp.concatenate(i_chunks, axis=1)

        scores_h, _, onehots = _topk8(all_s, iota_64, PK * PK)
        idx_h = []
        for sel in onehots:
            idx_h.append(jnp.sum(jnp.where(sel, all_i, 0), axis=1, keepdims=True))
        idx_h = jnp.concatenate(idx_h, axis=1)

        mx = jnp.max(scores_h, axis=1, keepdims=True)
        e = jnp.exp(scores_h - mx)
        w_parts.append(e / jnp.sum(e, axis=1, keepdims=True))
        idx_parts.append(idx_h)

    w_ref[...] = jnp.concatenate(w_parts, axis=1)
    idx_ref[...] = jnp.concatenate(idx_parts, axis=1)


def _phase_a(x2d, gamma, Wq, Kmat, interpret=False):
    grid = (N // TB,)
    return pl.pallas_call(
        _phase_a_body,
        grid=grid,
        in_specs=[
            pl.BlockSpec((TB, DIM), lambda i: (i, 0)),
            pl.BlockSpec((1, DIM), lambda i: (0, 0)),
            pl.BlockSpec((DIM, 2 * HEADS * DIM_KEY), lambda i: (0, 0)),
            pl.BlockSpec((2, HEADS, DIM_KEY, NUM_KEYS), lambda i: (0, 0, 0, 0)),
        ],
        out_specs=[
            pl.BlockSpec((TB, DIM), lambda i: (i, 0)),
            pl.BlockSpec((TB, HEADS * K), lambda i: (i, 0)),
            pl.BlockSpec((TB, HEADS * K), lambda i: (i, 0)),
        ],
        out_shape=[
            jax.ShapeDtypeStruct((N, DIM), jnp.float32),
            jax.ShapeDtypeStruct((N, HEADS * K), jnp.float32),
            jax.ShapeDtypeStruct((N, HEADS * K), jnp.int32),
        ],
        interpret=interpret,
    )(x2d, gamma.reshape(1, DIM), Wq, Kmat)


def kernel(x, gamma, Wq, keys_p, Wdown, Wup):
    x2d = x.reshape(N, DIM)
    Kmat = keys_p.transpose(2, 0, 3, 1)
    xn, w, idx = _phase_a(x2d, gamma, Wq, Kmat)
    out = _phase_b(xn, w, idx, Wdown, Wup)
    return out.reshape(1, N, DIM)


NC, NS = 2, 16
NW = NC * NS
TPW = N // NW
KH = HEADS * K


def _gelu_w(hvec, wvec):
    z = hvec * 0.7071067811865476
    az = jnp.abs(z)
    t = 1.0 / (1.0 + 0.3275911 * az)
    poly = t * (0.254829592 + t * (-0.284496736 + t * (1.421413741
               + t * (-1.453152027 + t * 1.061405429))))
    erf_abs = 1.0 - poly * jnp.exp(-az * az)
    erf = jnp.where(z < 0, -erf_abs, erf_abs)
    return hvec * 0.5 * (1.0 + erf) * wvec


def _phase_b_body(xn_hbm, w_hbm, idx_hbm, dn_hbm, up_hbm, out_hbm,
                  idx_v, w_v, xn_v, dn_rows, up_rows, out_v, g_smem,
                  sem_i, sem_d, sem_u, sem_x, sem_o):
    wid = lax.axis_index("s") * NC + lax.axis_index("c")
    base = wid * TPW

    pltpu.async_copy(idx_hbm.at[pl.ds(base, TPW)], idx_v, sem_i).wait()
    pltpu.async_copy(w_hbm.at[pl.ds(base, TPW)], w_v, sem_i).wait()

    lane = lax.iota(jnp.int32, 16)

    def token(t, carry):
        tok = base + t
        dnc = pltpu.async_copy(dn_hbm.at[idx_v.at[t]], dn_rows, sem_d)
        upc = pltpu.async_copy(up_hbm.at[idx_v.at[t]], up_rows, sem_u)
        pltpu.async_copy(xn_hbm.at[tok], xn_v, sem_x).wait()
        dnc.wait()

        h_scalars = []
        for kg in range(KH // 8):
            def dslice(s, accs, _kg=kg):
                xs = xn_v[pl.ds(s * 16, 16)]
                return tuple(
                    accs[j] + xs * dn_rows[_kg * 8 + j, pl.ds(s * 16, 16)]
                    for j in range(8))
            accs = lax.fori_loop(0, DIM // 16, dslice,
                                 tuple(jnp.zeros((16,), jnp.float32)
                                       for _ in range(8)))
            for j in range(8):
                h_scalars.append(jnp.sum(accs[j]))

        for kg in range(KH // 16):
            hv = jnp.zeros((16,), jnp.float32)
            for j in range(16):
                hv = jnp.where(lane == j, h_scalars[kg * 16 + j], hv)
            gv = _gelu_w(hv, w_v[t, pl.ds(kg * 16, 16)])
            for j in range(16):
                g_smem[kg * 16 + j] = jnp.sum(
                    jnp.where(lane == j, gv, 0.0))

        upc.wait()

        for chunk in range(3):
            def kstep(k, accs, _chunk=chunk):
                b = jnp.full((16,), g_smem[k], jnp.float32)
                return tuple(
                    accs[s] + b * up_rows[k, pl.ds((_chunk * 16 + s) * 16, 16)]
                    for s in range(16))
            accs = lax.fori_loop(0, KH, kstep,
                                 tuple(jnp.zeros((16,), jnp.float32)
                                       for _ in range(16)))
            for s in range(16):
                out_v[pl.ds((chunk * 16 + s) * 16, 16)] = accs[s]

        pltpu.async_copy(out_v, out_hbm.at[tok], sem_o).wait()
        return carry

    lax.fori_loop(0, TPW, token, 0)


def _phase_b(xn, w, idx, Wdown, Wup):
    mesh = plsc.VectorSubcoreMesh(core_axis_name="c", subcore_axis_name="s",
                                  num_cores=NC, num_subcores=NS)
    f = pl.kernel(
        _phase_b_body,
        out_type=jax.ShapeDtypeStruct((N, DIM), jnp.float32),
        mesh=mesh,
        compiler_params=pltpu.CompilerParams(needs_layout_passes=False),
        scratch_types=[
            pltpu.VMEM((TPW, KH), jnp.int32),
            pltpu.VMEM((TPW, KH), jnp.float32),
            pltpu.VMEM((DIM,), jnp.float32),
            pltpu.VMEM((KH, DIM), jnp.float32),
            pltpu.VMEM((KH, DIM), jnp.float32),
            pltpu.VMEM((DIM,), jnp.float32),
            pltpu.SMEM((KH,), jnp.float32),
            pltpu.SemaphoreType.DMA,
            pltpu.SemaphoreType.DMA,
            pltpu.SemaphoreType.DMA,
            pltpu.SemaphoreType.DMA,
            pltpu.SemaphoreType.DMA,
        ],
    )
    return f(xn, w, idx, Wdown, Wup)


# speedup vs baseline: 9.6583x; 1.2949x over previous
import functools

import jax
import jax.numpy as jnp
from jax import lax
from jax.experimental import pallas as pl
from jax.experimental.pallas import tpu as pltpu
from jax.experimental.pallas import tpu_sc as plsc

DIM = 768
HEADS = 8
NUM_KEYS = 256
DIM_KEY = DIM // 2
PK = 8
K = 8
N = 2048
TB = 256
NEG = -1e30


INT_MIN = -2 ** 31


def _mono(x):
    b = jax.lax.bitcast_convert_type(x, jnp.int32)
    return jnp.where(b < 0, b ^ 0x7FFFFFFF, b)


def _unmono(k):
    b = jnp.where(k < 0, k ^ 0x7FFFFFFF, k)
    return jax.lax.bitcast_convert_type(b, jnp.float32)


def _topk8_chain(vals):
    ms = [jnp.max(vals, axis=1, keepdims=True)]
    for _ in range(7):
        ms.append(jnp.max(jnp.where(vals < ms[-1], vals, NEG),
                          axis=1, keepdims=True))
    return ms


def _extract_idx(vals, ms, iota, bound):
    return jnp.concatenate(
        [jnp.min(jnp.where(vals == m, iota, bound), axis=1, keepdims=True)
         for m in ms], axis=1)


def _phase_a_body(x_ref, g_ref, wq_ref, km_ref, xn_ref, w_ref, idx_ref):
    x = x_ref[...]
    gamma = g_ref[...]
    nrm = jnp.sqrt(jnp.sum(x * x, axis=1, keepdims=True))
    nrm = jnp.maximum(nrm, 1e-12)
    xn = x / nrm * (DIM ** 0.5) * (gamma + 1.0)
    xn_ref[...] = xn

    q = jnp.dot(xn, wq_ref[...], preferred_element_type=jnp.float32)

    iota_k = lax.broadcasted_iota(jnp.int32, (TB, NUM_KEYS), 1)
    iota_64 = lax.broadcasted_iota(jnp.int32, (TB, PK * PK), 1)

    w_parts, idx_parts = [], []
    for h in range(HEADS):
        res = []
        for p in range(2):
            qs = q[:, p * (HEADS * DIM_KEY) + h * DIM_KEY:
                   p * (HEADS * DIM_KEY) + (h + 1) * DIM_KEY]
            sim = jnp.dot(qs, km_ref[p, h], preferred_element_type=jnp.float32)
            ms = _topk8_chain(sim)
            res.append((jnp.concatenate(ms, axis=1),
                        _extract_idx(sim, ms, iota_k, NUM_KEYS)))
        (sx, ix), (sy, iy) = res

        s_chunks, i_chunks = [], []
        for i in range(PK):
            s_chunks.append(sx[:, i:i + 1] + sy)
            i_chunks.append(ix[:, i:i + 1] * NUM_KEYS + iy)
        all_s = jnp.concatenate(s_chunks, axis=1)
        all_i = jnp.concatenate(i_chunks, axis=1)

        v2 = all_s
        sc_l, ix_l = [], []
        for _ in range(K):
            m = jnp.max(v2, axis=1, keepdims=True)
            am = jnp.min(jnp.where(v2 == m, iota_64, PK * PK),
                         axis=1, keepdims=True)
            sel = iota_64 == am
            sc_l.append(m)
            ix_l.append(jnp.max(jnp.where(sel, all_i, 0), axis=1, keepdims=True))
            v2 = jnp.where(sel, NEG, v2)
        scores_h = jnp.concatenate(sc_l, axis=1)
        idx_h = jnp.concatenate(ix_l, axis=1)

        mx = jnp.max(scores_h, axis=1, keepdims=True)
        e = jnp.exp(scores_h - mx)
        w_parts.append(e / jnp.sum(e, axis=1, keepdims=True))
        idx_parts.append(idx_h)

    w_ref[...] = jnp.concatenate(w_parts, axis=1)
    idx_ref[...] = jnp.concatenate(idx_parts, axis=1)


def _phase_a(x2d, gamma, Wq, Kmat, interpret=False):
    grid = (N // TB,)
    return pl.pallas_call(
        _phase_a_body,
        grid=grid,
        in_specs=[
            pl.BlockSpec((TB, DIM), lambda i: (i, 0)),
            pl.BlockSpec((1, DIM), lambda i: (0, 0)),
            pl.BlockSpec((DIM, 2 * HEADS * DIM_KEY), lambda i: (0, 0)),
            pl.BlockSpec((2, HEADS, DIM_KEY, NUM_KEYS), lambda i: (0, 0, 0, 0)),
        ],
        out_specs=[
            pl.BlockSpec((TB, DIM), lambda i: (i, 0)),
            pl.BlockSpec((TB, HEADS * K), lambda i: (i, 0)),
            pl.BlockSpec((TB, HEADS * K), lambda i: (i, 0)),
        ],
        out_shape=[
            jax.ShapeDtypeStruct((N, DIM), jnp.float32),
            jax.ShapeDtypeStruct((N, HEADS * K), jnp.float32),
            jax.ShapeDtypeStruct((N, HEADS * K), jnp.int32),
        ],
        interpret=interpret,
    )(x2d, gamma.reshape(1, DIM), Wq, Kmat)


def kernel(x, gamma, Wq, keys_p, Wdown, Wup):
    x2d = x.reshape(N, DIM)
    Kmat = keys_p.transpose(2, 0, 3, 1)
    xn, w, idx = _phase_a(x2d, gamma, Wq, Kmat)
    out = _phase_b(xn, w, idx, Wdown, Wup)
    return out.reshape(1, N, DIM)


NC, NS = 2, 16
NW = NC * NS
TPW = N // NW
KH = HEADS * K


def _gelu_w(hvec, wvec):
    z = hvec * 0.7071067811865476
    az = jnp.abs(z)
    t = 1.0 / (1.0 + 0.3275911 * az)
    poly = t * (0.254829592 + t * (-0.284496736 + t * (1.421413741
               + t * (-1.453152027 + t * 1.061405429))))
    erf_abs = 1.0 - poly * jnp.exp(-az * az)
    erf = jnp.where(z < 0, -erf_abs, erf_abs)
    return hvec * 0.5 * (1.0 + erf) * wvec


def _phase_b_body(xn_hbm, w_hbm, idx_hbm, dn_hbm, up_hbm, out_hbm,
                  idx_v, w_v, xn_v, dn_rows, up_rows, out_v, g_smem,
                  sem_i, sem_d, sem_u, sem_x, sem_o):
    wid = lax.axis_index("s") * NC + lax.axis_index("c")
    base = wid * TPW

    pltpu.async_copy(idx_hbm.at[pl.ds(base, TPW)], idx_v, sem_i).wait()
    pltpu.async_copy(w_hbm.at[pl.ds(base, TPW)], w_v, sem_i).wait()

    lane = lax.iota(jnp.int32, 16)

    pltpu.async_copy(dn_hbm.at[idx_v.at[0]], dn_rows, sem_d)
    pltpu.async_copy(up_hbm.at[idx_v.at[0]], up_rows, sem_u)
    pltpu.async_copy(xn_hbm.at[base], xn_v.at[0], sem_x)

    def token(t, carry):
        tok = base + t
        par = lax.rem(t, 2)
        pltpu.make_async_copy(xn_hbm.at[tok], xn_v.at[par], sem_x).wait()
        pltpu.make_async_copy(dn_hbm.at[idx_v.at[t]], dn_rows, sem_d).wait()

        h_scalars = []
        for kg in range(KH // 8):
            def dslice(s, accs, _kg=kg, _par=par):
                xs = xn_v[_par, pl.ds(s * 16, 16)]
                return tuple(
                    accs[j] + xs * dn_rows[_kg * 8 + j, pl.ds(s * 16, 16)]
                    for j in range(8))
            accs = lax.fori_loop(0, DIM // 16, dslice,
                                 tuple(jnp.zeros((16,), jnp.float32)
                                       for _ in range(8)))
            for j in range(8):
                h_scalars.append(jnp.sum(accs[j]))

        @pl.when(t < TPW - 1)
        def _():
            pltpu.async_copy(dn_hbm.at[idx_v.at[t + 1]], dn_rows, sem_d)
            pltpu.async_copy(xn_hbm.at[tok + 1], xn_v.at[1 - par], sem_x)

        for kg in range(KH // 16):
            hv = jnp.zeros((16,), jnp.float32)
            for j in range(16):
                hv = jnp.where(lane == j, h_scalars[kg * 16 + j], hv)
            gv = _gelu_w(hv, w_v[t, pl.ds(kg * 16, 16)])
            for j in range(16):
                g_smem[kg * 16 + j] = jnp.sum(
                    jnp.where(lane == j, gv, 0.0))

        pltpu.make_async_copy(up_hbm.at[idx_v.at[t]], up_rows, sem_u).wait()

        for chunk in range(3):
            def kstep(k, accs, _chunk=chunk):
                b = jnp.full((16,), g_smem[k], jnp.float32)
                return tuple(
                    accs[s] + b * up_rows[k, pl.ds((_chunk * 16 + s) * 16, 16)]
                    for s in range(16))
            accs = lax.fori_loop(0, KH, kstep,
                                 tuple(jnp.zeros((16,), jnp.float32)
                                       for _ in range(16)))
            for s in range(16):
                out_v[pl.ds((chunk * 16 + s) * 16, 16)] = accs[s]

        @pl.when(t < TPW - 1)
        def _():
            pltpu.async_copy(up_hbm.at[idx_v.at[t + 1]], up_rows, sem_u)

        pltpu.async_copy(out_v, out_hbm.at[tok], sem_o).wait()
        return carry

    lax.fori_loop(0, TPW, token, 0)


def _phase_b(xn, w, idx, Wdown, Wup):
    mesh = plsc.VectorSubcoreMesh(core_axis_name="c", subcore_axis_name="s",
                                  num_cores=NC, num_subcores=NS)
    f = pl.kernel(
        _phase_b_body,
        out_type=jax.ShapeDtypeStruct((N, DIM), jnp.float32),
        mesh=mesh,
        compiler_params=pltpu.CompilerParams(needs_layout_passes=False),
        scratch_types=[
            pltpu.VMEM((TPW, KH), jnp.int32),
            pltpu.VMEM((TPW, KH), jnp.float32),
            pltpu.VMEM((2, DIM), jnp.float32),
            pltpu.VMEM((KH, DIM), jnp.float32),
            pltpu.VMEM((KH, DIM), jnp.float32),
            pltpu.VMEM((DIM,), jnp.float32),
            pltpu.SMEM((KH,), jnp.float32),
            pltpu.SemaphoreType.DMA,
            pltpu.SemaphoreType.DMA,
            pltpu.SemaphoreType.DMA,
            pltpu.SemaphoreType.DMA,
            pltpu.SemaphoreType.DMA,
        ],
    )
    return f(xn, w, idx, Wdown, Wup)


# speedup vs baseline: 10.2298x; 1.0592x over previous
import functools

import jax
import jax.numpy as jnp
from jax import lax
from jax.experimental import pallas as pl
from jax.experimental.pallas import tpu as pltpu
from jax.experimental.pallas import tpu_sc as plsc

DIM = 768
HEADS = 8
NUM_KEYS = 256
DIM_KEY = DIM // 2
PK = 8
K = 8
N = 2048
TB = 256
NEG = -1e30


INT_MIN = -2 ** 31


def _mono(x):
    b = jax.lax.bitcast_convert_type(x, jnp.int32)
    return jnp.where(b < 0, b ^ 0x7FFFFFFF, b)


def _unmono(k):
    b = jnp.where(k < 0, k ^ 0x7FFFFFFF, k)
    return jax.lax.bitcast_convert_type(b, jnp.float32)


def _topk8_chain(vals):
    ms = [jnp.max(vals, axis=1, keepdims=True)]
    for _ in range(7):
        ms.append(jnp.max(jnp.where(vals < ms[-1], vals, NEG),
                          axis=1, keepdims=True))
    return ms


def _extract_idx(vals, ms, iota, bound):
    return jnp.concatenate(
        [jnp.min(jnp.where(vals == m, iota, bound), axis=1, keepdims=True)
         for m in ms], axis=1)


def _phase_a_body(x_ref, g_ref, wq_ref, km_ref, xn_ref, w_ref, idx_ref):
    x = x_ref[...]
    gamma = g_ref[...]
    nrm = jnp.sqrt(jnp.sum(x * x, axis=1, keepdims=True))
    nrm = jnp.maximum(nrm, 1e-12)
    xn = x / nrm * (DIM ** 0.5) * (gamma + 1.0)
    xn_ref[...] = xn

    q = jnp.dot(xn, wq_ref[...], preferred_element_type=jnp.float32)

    iota_k = lax.broadcasted_iota(jnp.int32, (TB, NUM_KEYS), 1)
    iota_64 = lax.broadcasted_iota(jnp.int32, (TB, PK * PK), 1)

    w_parts, idx_parts = [], []
    for h in range(HEADS):
        res = []
        for p in range(2):
            qs = q[:, p * (HEADS * DIM_KEY) + h * DIM_KEY:
                   p * (HEADS * DIM_KEY) + (h + 1) * DIM_KEY]
            sim = jnp.dot(qs, km_ref[p, h], preferred_element_type=jnp.float32)
            ms = _topk8_chain(sim)
            res.append((jnp.concatenate(ms, axis=1),
                        _extract_idx(sim, ms, iota_k, NUM_KEYS)))
        (sx, ix), (sy, iy) = res

        s_chunks, i_chunks = [], []
        for i in range(PK):
            s_chunks.append(sx[:, i:i + 1] + sy)
            i_chunks.append(ix[:, i:i + 1] * NUM_KEYS + iy)
        all_s = jnp.concatenate(s_chunks, axis=1)
        all_i = jnp.concatenate(i_chunks, axis=1)

        v2 = all_s
        sc_l, ix_l = [], []
        for _ in range(K):
            m = jnp.max(v2, axis=1, keepdims=True)
            am = jnp.min(jnp.where(v2 == m, iota_64, PK * PK),
                         axis=1, keepdims=True)
            sel = iota_64 == am
            sc_l.append(m)
            ix_l.append(jnp.max(jnp.where(sel, all_i, 0), axis=1, keepdims=True))
            v2 = jnp.where(sel, NEG, v2)
        scores_h = jnp.concatenate(sc_l, axis=1)
        idx_h = jnp.concatenate(ix_l, axis=1)

        mx = jnp.max(scores_h, axis=1, keepdims=True)
        e = jnp.exp(scores_h - mx)
        w_parts.append(e / jnp.sum(e, axis=1, keepdims=True))
        idx_parts.append(idx_h)

    w_ref[...] = jnp.concatenate(w_parts, axis=1)
    idx_ref[...] = jnp.concatenate(idx_parts, axis=1)


def _phase_a(x2d, gamma, Wq, Kmat, interpret=False):
    grid = (x2d.shape[0] // TB,)
    nt = x2d.shape[0]
    return pl.pallas_call(
        _phase_a_body,
        grid=grid,
        in_specs=[
            pl.BlockSpec((TB, DIM), lambda i: (i, 0)),
            pl.BlockSpec((1, DIM), lambda i: (0, 0)),
            pl.BlockSpec((DIM, 2 * HEADS * DIM_KEY), lambda i: (0, 0)),
            pl.BlockSpec((2, HEADS, DIM_KEY, NUM_KEYS), lambda i: (0, 0, 0, 0)),
        ],
        out_specs=[
            pl.BlockSpec((TB, DIM), lambda i: (i, 0)),
            pl.BlockSpec((TB, HEADS * K), lambda i: (i, 0)),
            pl.BlockSpec((TB, HEADS * K), lambda i: (i, 0)),
        ],
        out_shape=[
            jax.ShapeDtypeStruct((nt, DIM), jnp.float32),
            jax.ShapeDtypeStruct((nt, HEADS * K), jnp.float32),
            jax.ShapeDtypeStruct((nt, HEADS * K), jnp.int32),
        ],
        interpret=interpret,
    )(x2d, gamma.reshape(1, DIM), Wq, Kmat)


NB = 4
TBB = N // NB


def kernel(x, gamma, Wq, keys_p, Wdown, Wup):
    x2d = x.reshape(N, DIM)
    Kmat = keys_p.transpose(2, 0, 3, 1)
    outs = []
    for i in range(NB):
        xn, w, idx = _phase_a(x2d[i * TBB:(i + 1) * TBB], gamma, Wq, Kmat)
        outs.append(_phase_b(xn, w, idx, Wdown, Wup))
    return jnp.concatenate(outs, axis=0).reshape(1, N, DIM)


NC, NS = 2, 16
NW = NC * NS
TPW = TBB // NW
KH = HEADS * K


def _gelu_w(hvec, wvec):
    z = hvec * 0.7071067811865476
    az = jnp.abs(z)
    t = 1.0 / (1.0 + 0.3275911 * az)
    poly = t * (0.254829592 + t * (-0.284496736 + t * (1.421413741
               + t * (-1.453152027 + t * 1.061405429))))
    erf_abs = 1.0 - poly * jnp.exp(-az * az)
    erf = jnp.where(z < 0, -erf_abs, erf_abs)
    return hvec * 0.5 * (1.0 + erf) * wvec


def _phase_b_body(xn_hbm, w_hbm, idx_hbm, dn_hbm, up_hbm, out_hbm,
                  idx_v, w_v, xn_v, dn_rows, up_rows, out_v, g_smem,
                  sem_i, sem_d, sem_u, sem_x, sem_o):
    wid = lax.axis_index("s") * NC + lax.axis_index("c")
    base = wid * TPW

    pltpu.async_copy(idx_hbm.at[pl.ds(base, TPW)], idx_v, sem_i).wait()
    pltpu.async_copy(w_hbm.at[pl.ds(base, TPW)], w_v, sem_i).wait()

    lane = lax.iota(jnp.int32, 16)

    pltpu.async_copy(dn_hbm.at[idx_v.at[0]], dn_rows, sem_d)
    pltpu.async_copy(up_hbm.at[idx_v.at[0]], up_rows, sem_u)
    pltpu.async_copy(xn_hbm.at[base], xn_v.at[0], sem_x)

    def token(t, carry):
        tok = base + t
        par = lax.rem(t, 2)
        pltpu.make_async_copy(xn_hbm.at[tok], xn_v.at[par], sem_x).wait()
        pltpu.make_async_copy(dn_hbm.at[idx_v.at[t]], dn_rows, sem_d).wait()

        h_scalars = []
        for kg in range(KH // 8):
            def dslice(s, accs, _kg=kg, _par=par):
                xs = xn_v[_par, pl.ds(s * 16, 16)]
                return tuple(
                    accs[j] + xs * dn_rows[_kg * 8 + j, pl.ds(s * 16, 16)]
                    for j in range(8))
            accs = lax.fori_loop(0, DIM // 16, dslice,
                                 tuple(jnp.zeros((16,), jnp.float32)
                                       for _ in range(8)))
            for j in range(8):
                h_scalars.append(jnp.sum(accs[j]))

        @pl.when(t < TPW - 1)
        def _():
            pltpu.async_copy(dn_hbm.at[idx_v.at[t + 1]], dn_rows, sem_d)
            pltpu.async_copy(xn_hbm.at[tok + 1], xn_v.at[1 - par], sem_x)

        for kg in range(KH // 16):
            hv = jnp.zeros((16,), jnp.float32)
            for j in range(16):
                hv = jnp.where(lane == j, h_scalars[kg * 16 + j], hv)
            gv = _gelu_w(hv, w_v[t, pl.ds(kg * 16, 16)])
            for j in range(16):
                g_smem[kg * 16 + j] = jnp.sum(
                    jnp.where(lane == j, gv, 0.0))

        pltpu.make_async_copy(up_hbm.at[idx_v.at[t]], up_rows, sem_u).wait()

        for chunk in range(3):
            def kstep(k, accs, _chunk=chunk):
                b = jnp.full((16,), g_smem[k], jnp.float32)
                return tuple(
                    accs[s] + b * up_rows[k, pl.ds((_chunk * 16 + s) * 16, 16)]
                    for s in range(16))
            accs = lax.fori_loop(0, KH, kstep,
                                 tuple(jnp.zeros((16,), jnp.float32)
                                       for _ in range(16)))
            for s in range(16):
                out_v[pl.ds((chunk * 16 + s) * 16, 16)] = accs[s]

        @pl.when(t < TPW - 1)
        def _():
            pltpu.async_copy(up_hbm.at[idx_v.at[t + 1]], up_rows, sem_u)

        pltpu.async_copy(out_v, out_hbm.at[tok], sem_o).wait()
        return carry

    lax.fori_loop(0, TPW, token, 0)


def _phase_b(xn, w, idx, Wdown, Wup):
    mesh = plsc.VectorSubcoreMesh(core_axis_name="c", subcore_axis_name="s",
                                  num_cores=NC, num_subcores=NS)
    f = pl.kernel(
        _phase_b_body,
        out_type=jax.ShapeDtypeStruct((TBB, DIM), jnp.float32),
        mesh=mesh,
        compiler_params=pltpu.CompilerParams(needs_layout_passes=False),
        scratch_types=[
            pltpu.VMEM((TPW, KH), jnp.int32),
            pltpu.VMEM((TPW, KH), jnp.float32),
            pltpu.VMEM((2, DIM), jnp.float32),
            pltpu.VMEM((KH, DIM), jnp.float32),
            pltpu.VMEM((KH, DIM), jnp.float32),
            pltpu.VMEM((DIM,), jnp.float32),
            pltpu.SMEM((KH,), jnp.float32),
            pltpu.SemaphoreType.DMA,
            pltpu.SemaphoreType.DMA,
            pltpu.SemaphoreType.DMA,
            pltpu.SemaphoreType.DMA,
            pltpu.SemaphoreType.DMA,
        ],
    )
    return f(xn, w, idx, Wdown, Wup)


# speedup vs baseline: 16.5295x; 1.6158x over previous
import functools

import jax
import jax.numpy as jnp
from jax import lax
from jax.experimental import pallas as pl
from jax.experimental.pallas import tpu as pltpu
from jax.experimental.pallas import tpu_sc as plsc

DIM = 768
HEADS = 8
NUM_KEYS = 256
DIM_KEY = DIM // 2
PK = 8
K = 8
N = 2048
TB = 512
NEG = -1e30


INT_MIN = -2 ** 31


def _mono(x):
    b = jax.lax.bitcast_convert_type(x, jnp.int32)
    return jnp.where(b < 0, b ^ 0x7FFFFFFF, b)


def _unmono(k):
    b = jnp.where(k < 0, k ^ 0x7FFFFFFF, k)
    return jax.lax.bitcast_convert_type(b, jnp.float32)


def _topk8_chain(vals):
    ms = [jnp.max(vals, axis=1, keepdims=True)]
    for _ in range(7):
        ms.append(jnp.max(jnp.where(vals < ms[-1], vals, NEG),
                          axis=1, keepdims=True))
    return ms


def _extract_idx(vals, ms, iota, bound):
    return jnp.concatenate(
        [jnp.min(jnp.where(vals == m, iota, bound), axis=1, keepdims=True)
         for m in ms], axis=1)


def _phase_a_body(x_ref, g_ref, wq_ref, km_ref, xn_ref, w_ref, idx_ref):
    x = x_ref[...]
    gamma = g_ref[...]
    nrm = jnp.sqrt(jnp.sum(x * x, axis=1, keepdims=True))
    nrm = jnp.maximum(nrm, 1e-12)
    xn = x / nrm * (DIM ** 0.5) * (gamma + 1.0)
    xn_ref[...] = xn

    q = jnp.dot(xn, wq_ref[...], preferred_element_type=jnp.float32)

    iota_k = lax.broadcasted_iota(jnp.int32, (TB, NUM_KEYS), 1)
    iota_64 = lax.broadcasted_iota(jnp.int32, (TB, PK * PK), 1)

    w_parts, idx_parts = [], []
    for h in range(HEADS):
        res = []
        for p in range(2):
            qs = q[:, p * (HEADS * DIM_KEY) + h * DIM_KEY:
                   p * (HEADS * DIM_KEY) + (h + 1) * DIM_KEY]
            sim = jnp.dot(qs, km_ref[p, h], preferred_element_type=jnp.float32)
            ms = _topk8_chain(sim)
            res.append((jnp.concatenate(ms, axis=1),
                        _extract_idx(sim, ms, iota_k, NUM_KEYS)))
        (sx, ix), (sy, iy) = res

        s_chunks, i_chunks = [], []
        for i in range(PK):
            s_chunks.append(sx[:, i:i + 1] + sy)
            i_chunks.append(ix[:, i:i + 1] * NUM_KEYS + iy)
        all_s = jnp.concatenate(s_chunks, axis=1)
        all_i = jnp.concatenate(i_chunks, axis=1)

        v2 = all_s
        sc_l, ix_l = [], []
        for _ in range(K):
            m = jnp.max(v2, axis=1, keepdims=True)
            am = jnp.min(jnp.where(v2 == m, iota_64, PK * PK),
                         axis=1, keepdims=True)
            sel = iota_64 == am
            sc_l.append(m)
            ix_l.append(jnp.max(jnp.where(sel, all_i, 0), axis=1, keepdims=True))
            v2 = jnp.where(sel, NEG, v2)
        scores_h = jnp.concatenate(sc_l, axis=1)
        idx_h = jnp.concatenate(ix_l, axis=1)

        mx = jnp.max(scores_h, axis=1, keepdims=True)
        e = jnp.exp(scores_h - mx)
        w_parts.append(e / jnp.sum(e, axis=1, keepdims=True))
        idx_parts.append(idx_h)

    w_ref[...] = jnp.concatenate(w_parts, axis=1)
    idx_ref[...] = jnp.concatenate(idx_parts, axis=1)


def _phase_a(x2d, gamma, Wq, Kmat, interpret=False):
    grid = (x2d.shape[0] // TB,)
    nt = x2d.shape[0]
    return pl.pallas_call(
        _phase_a_body,
        grid=grid,
        in_specs=[
            pl.BlockSpec((TB, DIM), lambda i: (i, 0)),
            pl.BlockSpec((1, DIM), lambda i: (0, 0)),
            pl.BlockSpec((DIM, 2 * HEADS * DIM_KEY), lambda i: (0, 0)),
            pl.BlockSpec((2, HEADS, DIM_KEY, NUM_KEYS), lambda i: (0, 0, 0, 0)),
        ],
        out_specs=[
            pl.BlockSpec((TB, DIM), lambda i: (i, 0)),
            pl.BlockSpec((TB, HEADS * K), lambda i: (i, 0)),
            pl.BlockSpec((TB, HEADS * K), lambda i: (i, 0)),
        ],
        out_shape=[
            jax.ShapeDtypeStruct((nt, DIM), jnp.float32),
            jax.ShapeDtypeStruct((nt, HEADS * K), jnp.float32),
            jax.ShapeDtypeStruct((nt, HEADS * K), jnp.int32),
        ],
        interpret=interpret,
    )(x2d, gamma.reshape(1, DIM), Wq, Kmat)


NB = 4
TBB = N // NB


def kernel(x, gamma, Wq, keys_p, Wdown, Wup):
    x2d = x.reshape(N, DIM)
    Kmat = keys_p.transpose(2, 0, 3, 1)
    outs = []
    for i in range(NB):
        xn, w, idx = _phase_a(x2d[i * TBB:(i + 1) * TBB], gamma, Wq, Kmat)
        outs.append(_phase_b(xn, w, idx, Wdown, Wup))
    return jnp.concatenate(outs, axis=0).reshape(1, N, DIM)


NC, NS = 2, 16
NW = NC * NS
TPW = TBB // NW
KH = HEADS * K


def _gelu_w(hvec, wvec):
    z = hvec * 0.7071067811865476
    az = jnp.abs(z)
    t = 1.0 / (1.0 + 0.3275911 * az)
    poly = t * (0.254829592 + t * (-0.284496736 + t * (1.421413741
               + t * (-1.453152027 + t * 1.061405429))))
    erf_abs = 1.0 - poly * jnp.exp(-az * az)
    erf = jnp.where(z < 0, -erf_abs, erf_abs)
    return hvec * 0.5 * (1.0 + erf) * wvec


def _phase_b_body(xn_hbm, w_hbm, idx_hbm, dn_hbm, up_hbm, out_hbm,
                  idx_v, w_v, xn_v, dn_rows, up_rows, out_v, g_smem,
                  sem_i, sem_d, sem_u, sem_x, sem_o):
    wid = lax.axis_index("s") * NC + lax.axis_index("c")
    base = wid * TPW

    pltpu.async_copy(idx_hbm.at[pl.ds(base, TPW)], idx_v, sem_i).wait()
    pltpu.async_copy(w_hbm.at[pl.ds(base, TPW)], w_v, sem_i).wait()

    lane = lax.iota(jnp.int32, 16)

    pltpu.async_copy(dn_hbm.at[idx_v.at[0]], dn_rows, sem_d)
    pltpu.async_copy(up_hbm.at[idx_v.at[0]], up_rows, sem_u)
    pltpu.async_copy(xn_hbm.at[base], xn_v.at[0], sem_x)

    def token(t, carry):
        tok = base + t
        par = lax.rem(t, 2)
        pltpu.make_async_copy(xn_hbm.at[tok], xn_v.at[par], sem_x).wait()
        pltpu.make_async_copy(dn_hbm.at[idx_v.at[t]], dn_rows, sem_d).wait()

        h_scalars = []
        for kg in range(KH // 8):
            def dslice(s, accs, _kg=kg, _par=par):
                xs = xn_v[_par, pl.ds(s * 16, 16)]
                return tuple(
                    accs[j] + xs * dn_rows[_kg * 8 + j, pl.ds(s * 16, 16)]
                    for j in range(8))
            accs = lax.fori_loop(0, DIM // 16, dslice,
                                 tuple(jnp.zeros((16,), jnp.float32)
                                       for _ in range(8)))
            for j in range(8):
                h_scalars.append(jnp.sum(accs[j]))

        @pl.when(t < TPW - 1)
        def _():
            pltpu.async_copy(dn_hbm.at[idx_v.at[t + 1]], dn_rows, sem_d)
            pltpu.async_copy(xn_hbm.at[tok + 1], xn_v.at[1 - par], sem_x)

        for kg in range(KH // 16):
            hv = jnp.zeros((16,), jnp.float32)
            for j in range(16):
                hv = jnp.where(lane == j, h_scalars[kg * 16 + j], hv)
            gv = _gelu_w(hv, w_v[t, pl.ds(kg * 16, 16)])
            for j in range(16):
                g_smem[kg * 16 + j] = jnp.sum(
                    jnp.where(lane == j, gv, 0.0))

        pltpu.make_async_copy(up_hbm.at[idx_v.at[t]], up_rows, sem_u).wait()

        for chunk in range(3):
            def kstep(k, accs, _chunk=chunk):
                b = jnp.full((16,), g_smem[k], jnp.float32)
                return tuple(
                    accs[s] + b * up_rows[k, pl.ds((_chunk * 16 + s) * 16, 16)]
                    for s in range(16))
            accs = lax.fori_loop(0, KH, kstep,
                                 tuple(jnp.zeros((16,), jnp.float32)
                                       for _ in range(16)))
            for s in range(16):
                out_v[pl.ds((chunk * 16 + s) * 16, 16)] = accs[s]

        @pl.when(t < TPW - 1)
        def _():
            pltpu.async_copy(up_hbm.at[idx_v.at[t + 1]], up_rows, sem_u)

        pltpu.async_copy(out_v, out_hbm.at[tok], sem_o).wait()
        return carry

    lax.fori_loop(0, TPW, token, 0)


def _phase_b(xn, w, idx, Wdown, Wup):
    mesh = plsc.VectorSubcoreMesh(core_axis_name="c", subcore_axis_name="s",
                                  num_cores=NC, num_subcores=NS)
    f = pl.kernel(
        _phase_b_body,
        out_type=jax.ShapeDtypeStruct((TBB, DIM), jnp.float32),
        mesh=mesh,
        compiler_params=pltpu.CompilerParams(needs_layout_passes=False),
        scratch_types=[
            pltpu.VMEM((TPW, KH), jnp.int32),
            pltpu.VMEM((TPW, KH), jnp.float32),
            pltpu.VMEM((2, DIM), jnp.float32),
            pltpu.VMEM((KH, DIM), jnp.float32),
            pltpu.VMEM((KH, DIM), jnp.float32),
            pltpu.VMEM((DIM,), jnp.float32),
            pltpu.SMEM((KH,), jnp.float32),
            pltpu.SemaphoreType.DMA,
            pltpu.SemaphoreType.DMA,
            pltpu.SemaphoreType.DMA,
            pltpu.SemaphoreType.DMA,
            pltpu.SemaphoreType.DMA,
        ],
    )
    return f(xn, w, idx, Wdown, Wup)


# speedup vs baseline: 17.9221x; 1.0842x over previous
import functools

import jax
import jax.numpy as jnp
from jax import lax
from jax.experimental import pallas as pl
from jax.experimental.pallas import tpu as pltpu
from jax.experimental.pallas import tpu_sc as plsc

DIM = 768
HEADS = 8
NUM_KEYS = 256
NUM_EXPERTS = NUM_KEYS * NUM_KEYS
DIM_KEY = DIM // 2
PK = 8
K = 8
N = 2048
TB = 512
NEG = -1e30


def _topk8_mxu(v, ltstrict):
    sc, firsts = [], []
    for _ in range(8):
        m = jnp.max(v, axis=1, keepdims=True)
        sc.append(m)
        sel = jnp.where(v == m, 1.0, 0.0)
        pre = jnp.dot(sel, ltstrict)
        first = sel * jnp.maximum(1.0 - pre, 0.0)
        firsts.append(first)
        v = v - first * 1e30
    return jnp.concatenate(sc, axis=1), firsts


def _phase_a_body(x_ref, g_ref, wq_ref, km_ref, xn_ref, w_ref, idx_ref):
    x = x_ref[...]
    gamma = g_ref[...]
    nrm = jnp.sqrt(jnp.sum(x * x, axis=1, keepdims=True))
    nrm = jnp.maximum(nrm, 1e-12)
    xn = x / nrm * (DIM ** 0.5) * (gamma + 1.0)
    xn_ref[...] = xn

    q = jnp.dot(xn, wq_ref[...], preferred_element_type=jnp.float32)

    hi = jax.lax.Precision.HIGHEST
    io_k = lax.broadcasted_iota(
        jnp.int32, (NUM_KEYS, 1), 0).astype(jnp.float32)
    io_r = lax.broadcasted_iota(jnp.int32, (NUM_KEYS, NUM_KEYS), 0)
    io_c = lax.broadcasted_iota(jnp.int32, (NUM_KEYS, NUM_KEYS), 1)
    lt_k = jnp.where(io_c < io_r, 1.0, 0.0)
    lt64 = lt_k[:PK * PK, :PK * PK]
    ones64 = jnp.zeros((PK * PK, 1), jnp.float32) + 1.0
    r64 = lax.broadcasted_iota(jnp.int32, (PK, PK * PK), 0)
    c64 = lax.broadcasted_iota(jnp.int32, (PK, PK * PK), 1)
    e1 = jnp.where(c64 // PK == r64, 1.0, 0.0)
    e2 = jnp.where(c64 % PK == r64, 1.0, 0.0)

    w_parts, idx_parts = [], []
    for h in range(HEADS):
        res = []
        for p in range(2):
            qs = q[:, p * (HEADS * DIM_KEY) + h * DIM_KEY:
                   p * (HEADS * DIM_KEY) + (h + 1) * DIM_KEY]
            sim = jnp.dot(qs, km_ref[p, h], preferred_element_type=jnp.float32)
            sxy, firsts = _topk8_mxu(sim, lt_k)
            ixy = jnp.concatenate(
                [jnp.dot(f, io_k) for f in firsts], axis=1)
            res.append((sxy, ixy))
        (sx, ix), (sy, iy) = res

        all_s = (jnp.dot(sx, e1, precision=hi)
                 + jnp.dot(sy, e2, precision=hi))
        axf = jnp.dot(ix, e1)
        ayf = jnp.dot(iy, e2)

        scores_h, firsts2 = _topk8_mxu(all_s, lt64)
        exf = jnp.concatenate(
            [jnp.dot(f * axf, ones64) for f in firsts2], axis=1)
        eyf = jnp.concatenate(
            [jnp.dot(f * ayf, ones64) for f in firsts2], axis=1)
        idx_h = exf * float(NUM_KEYS) + eyf

        mx = jnp.max(scores_h, axis=1, keepdims=True)
        e = jnp.exp(scores_h - mx)
        w_parts.append(e / jnp.sum(e, axis=1, keepdims=True))
        idx_parts.append(idx_h)

    w_ref[...] = jnp.concatenate(w_parts, axis=1)
    idx_ref[...] = jnp.minimum(
        jnp.concatenate(idx_parts, axis=1),
        float(NUM_EXPERTS - 1)).astype(jnp.int32)


def _phase_a(x2d, gamma, Wq, Kmat, interpret=False):
    grid = (x2d.shape[0] // TB,)
    nt = x2d.shape[0]
    return pl.pallas_call(
        _phase_a_body,
        grid=grid,
        in_specs=[
            pl.BlockSpec((TB, DIM), lambda i: (i, 0)),
            pl.BlockSpec((1, DIM), lambda i: (0, 0)),
            pl.BlockSpec((DIM, 2 * HEADS * DIM_KEY), lambda i: (0, 0)),
            pl.BlockSpec((2, HEADS, DIM_KEY, NUM_KEYS), lambda i: (0, 0, 0, 0)),
        ],
        out_specs=[
            pl.BlockSpec((TB, DIM), lambda i: (i, 0)),
            pl.BlockSpec((TB, HEADS * K), lambda i: (i, 0)),
            pl.BlockSpec((TB, HEADS * K), lambda i: (i, 0)),
        ],
        out_shape=[
            jax.ShapeDtypeStruct((nt, DIM), jnp.float32),
            jax.ShapeDtypeStruct((nt, HEADS * K), jnp.float32),
            jax.ShapeDtypeStruct((nt, HEADS * K), jnp.int32),
        ],
        interpret=interpret,
    )(x2d, gamma.reshape(1, DIM), Wq, Kmat)


NB = 4
TBB = N // NB


def kernel(x, gamma, Wq, keys_p, Wdown, Wup):
    x2d = x.reshape(N, DIM)
    Kmat = keys_p.transpose(2, 0, 3, 1)
    outs = []
    for i in range(NB):
        xn, w, idx = _phase_a(x2d[i * TBB:(i + 1) * TBB], gamma, Wq, Kmat)
        outs.append(_phase_b(xn, w, idx, Wdown, Wup))
    return jnp.concatenate(outs, axis=0).reshape(1, N, DIM)


NC, NS = 2, 16
NW = NC * NS
TPW = TBB // NW
KH = HEADS * K


def _gelu_w(hvec, wvec):
    z = hvec * 0.7071067811865476
    az = jnp.abs(z)
    t = 1.0 / (1.0 + 0.3275911 * az)
    poly = t * (0.254829592 + t * (-0.284496736 + t * (1.421413741
               + t * (-1.453152027 + t * 1.061405429))))
    erf_abs = 1.0 - poly * jnp.exp(-az * az)
    erf = jnp.where(z < 0, -erf_abs, erf_abs)
    return hvec * 0.5 * (1.0 + erf) * wvec


def _phase_b_body(xn_hbm, w_hbm, idx_hbm, dn_hbm, up_hbm, out_hbm,
                  idx_v, w_v, xn_v, dn_rows, up_rows, out_v, g_smem,
                  sem_i, sem_d, sem_u, sem_x, sem_o):
    wid = lax.axis_index("s") * NC + lax.axis_index("c")
    base = wid * TPW

    pltpu.async_copy(idx_hbm.at[pl.ds(base, TPW)], idx_v, sem_i).wait()
    pltpu.async_copy(w_hbm.at[pl.ds(base, TPW)], w_v, sem_i).wait()

    lane = lax.iota(jnp.int32, 16)

    pltpu.async_copy(dn_hbm.at[idx_v.at[0]], dn_rows, sem_d)
    pltpu.async_copy(up_hbm.at[idx_v.at[0]], up_rows, sem_u)
    pltpu.async_copy(xn_hbm.at[base], xn_v.at[0], sem_x)

    def token(t, carry):
        tok = base + t
        par = lax.rem(t, 2)
        pltpu.make_async_copy(xn_hbm.at[tok], xn_v.at[par], sem_x).wait()
        pltpu.make_async_copy(dn_hbm.at[idx_v.at[t]], dn_rows, sem_d).wait()

        h_scalars = []
        for kg in range(KH // 8):
            def dslice(s, accs, _kg=kg, _par=par):
                xs = xn_v[_par, pl.ds(s * 16, 16)]
                return tuple(
                    accs[j] + xs * dn_rows[_kg * 8 + j, pl.ds(s * 16, 16)]
                    for j in range(8))
            accs = lax.fori_loop(0, DIM // 16, dslice,
                                 tuple(jnp.zeros((16,), jnp.float32)
                                       for _ in range(8)))
            for j in range(8):
                h_scalars.append(jnp.sum(accs[j]))

        @pl.when(t < TPW - 1)
        def _():
            pltpu.async_copy(dn_hbm.at[idx_v.at[t + 1]], dn_rows, sem_d)
            pltpu.async_copy(xn_hbm.at[tok + 1], xn_v.at[1 - par], sem_x)

        for kg in range(KH // 16):
            hv = jnp.zeros((16,), jnp.float32)
            for j in range(16):
                hv = jnp.where(lane == j, h_scalars[kg * 16 + j], hv)
            gv = _gelu_w(hv, w_v[t, pl.ds(kg * 16, 16)])
            for j in range(16):
                g_smem[kg * 16 + j] = jnp.sum(
                    jnp.where(lane == j, gv, 0.0))

        pltpu.make_async_copy(up_hbm.at[idx_v.at[t]], up_rows, sem_u).wait()

        for chunk in range(3):
            def kstep(k, accs, _chunk=chunk):
                b = jnp.full((16,), g_smem[k], jnp.float32)
                return tuple(
                    accs[s] + b * up_rows[k, pl.ds((_chunk * 16 + s) * 16, 16)]
                    for s in range(16))
            accs = lax.fori_loop(0, KH, kstep,
                                 tuple(jnp.zeros((16,), jnp.float32)
                                       for _ in range(16)))
            for s in range(16):
                out_v[pl.ds((chunk * 16 + s) * 16, 16)] = accs[s]

        @pl.when(t < TPW - 1)
        def _():
            pltpu.async_copy(up_hbm.at[idx_v.at[t + 1]], up_rows, sem_u)

        pltpu.async_copy(out_v, out_hbm.at[tok], sem_o).wait()
        return carry

    lax.fori_loop(0, TPW, token, 0)


def _phase_b(xn, w, idx, Wdown, Wup):
    mesh = plsc.VectorSubcoreMesh(core_axis_name="c", subcore_axis_name="s",
                                  num_cores=NC, num_subcores=NS)
    f = pl.kernel(
        _phase_b_body,
        out_type=jax.ShapeDtypeStruct((TBB, DIM), jnp.float32),
        mesh=mesh,
        compiler_params=pltpu.CompilerParams(needs_layout_passes=False),
        scratch_types=[
            pltpu.VMEM((TPW, KH), jnp.int32),
            pltpu.VMEM((TPW, KH), jnp.float32),
            pltpu.VMEM((2, DIM), jnp.float32),
            pltpu.VMEM((KH, DIM), jnp.float32),
            pltpu.VMEM((KH, DIM), jnp.float32),
            pltpu.VMEM((DIM,), jnp.float32),
            pltpu.SMEM((KH,), jnp.float32),
            pltpu.SemaphoreType.DMA,
            pltpu.SemaphoreType.DMA,
            pltpu.SemaphoreType.DMA,
            pltpu.SemaphoreType.DMA,
            pltpu.SemaphoreType.DMA,
        ],
    )
    return f(xn, w, idx, Wdown, Wup)


# speedup vs baseline: 18.6046x; 1.0381x over previous
import functools

import jax
import jax.numpy as jnp
from jax import lax
from jax.experimental import pallas as pl
from jax.experimental.pallas import tpu as pltpu
from jax.experimental.pallas import tpu_sc as plsc

DIM = 768
HEADS = 8
NUM_KEYS = 256
NUM_EXPERTS = NUM_KEYS * NUM_KEYS
DIM_KEY = DIM // 2
PK = 8
K = 8
N = 2048
TB = 512
NEG = -1e30


def _topk8_mxu(v, ltstrict):
    sc, firsts = [], []
    for _ in range(8):
        m = jnp.max(v, axis=1, keepdims=True)
        sc.append(m)
        sel = jnp.where(v == m, 1.0, 0.0)
        pre = jnp.dot(sel, ltstrict)
        first = sel - jnp.minimum(sel, pre)
        firsts.append(first)
        v = v - first * 1e30
    return jnp.concatenate(sc, axis=1), firsts


def _phase_a_body(x_ref, g_ref, wq_ref, km_ref, xn_ref, w_ref, idx_ref):
    x = x_ref[...]
    gamma = g_ref[...]
    nrm = jnp.sqrt(jnp.sum(x * x, axis=1, keepdims=True))
    nrm = jnp.maximum(nrm, 1e-12)
    xn = x / nrm * (DIM ** 0.5) * (gamma + 1.0)
    xn_ref[...] = xn

    q = jnp.dot(xn, wq_ref[...], preferred_element_type=jnp.float32)

    hi = jax.lax.Precision.HIGHEST
    io_k = lax.broadcasted_iota(
        jnp.int32, (NUM_KEYS, 1), 0).astype(jnp.float32)
    io_r = lax.broadcasted_iota(jnp.int32, (NUM_KEYS, NUM_KEYS), 0)
    io_c = lax.broadcasted_iota(jnp.int32, (NUM_KEYS, NUM_KEYS), 1)
    lt_k = jnp.where(io_c < io_r, 1.0, 0.0)
    lt64 = lt_k[:PK * PK, :PK * PK]
    ones64 = jnp.zeros((PK * PK, 1), jnp.float32) + 1.0
    r64 = lax.broadcasted_iota(jnp.int32, (PK, PK * PK), 0)
    c64 = lax.broadcasted_iota(jnp.int32, (PK, PK * PK), 1)
    e1 = jnp.where(c64 // PK == r64, 1.0, 0.0)
    e2 = jnp.where(c64 % PK == r64, 1.0, 0.0)

    w_parts, idx_parts = [], []
    for h in range(HEADS):
        res = []
        for p in range(2):
            qs = q[:, p * (HEADS * DIM_KEY) + h * DIM_KEY:
                   p * (HEADS * DIM_KEY) + (h + 1) * DIM_KEY]
            sim = jnp.dot(qs, km_ref[p, h], preferred_element_type=jnp.float32)
            sxy, firsts = _topk8_mxu(sim, lt_k)
            ixy = jnp.concatenate(
                [jnp.dot(f, io_k) for f in firsts], axis=1)
            res.append((sxy, ixy))
        (sx, ix), (sy, iy) = res

        all_s = (jnp.dot(sx, e1, precision=hi)
                 + jnp.dot(sy, e2, precision=hi))
        axf = jnp.dot(ix, e1)
        ayf = jnp.dot(iy, e2)

        scores_h, firsts2 = _topk8_mxu(all_s, lt64)
        exf = jnp.concatenate(
            [jnp.dot(f * axf, ones64) for f in firsts2], axis=1)
        eyf = jnp.concatenate(
            [jnp.dot(f * ayf, ones64) for f in firsts2], axis=1)
        idx_h = exf * float(NUM_KEYS) + eyf

        mx = jnp.max(scores_h, axis=1, keepdims=True)
        e = jnp.exp(scores_h - mx)
        w_parts.append(e / jnp.sum(e, axis=1, keepdims=True))
        idx_parts.append(idx_h)

    w_ref[...] = jnp.concatenate(w_parts, axis=1)
    idx_ref[...] = jnp.minimum(
        jnp.concatenate(idx_parts, axis=1),
        float(NUM_EXPERTS - 1)).astype(jnp.int32)


def _phase_a(x2d, gamma, Wq, Kmat, interpret=False):
    grid = (x2d.shape[0] // TB,)
    nt = x2d.shape[0]
    return pl.pallas_call(
        _phase_a_body,
        grid=grid,
        in_specs=[
            pl.BlockSpec((TB, DIM), lambda i: (i, 0)),
            pl.BlockSpec((1, DIM), lambda i: (0, 0)),
            pl.BlockSpec((DIM, 2 * HEADS * DIM_KEY), lambda i: (0, 0)),
            pl.BlockSpec((2, HEADS, DIM_KEY, NUM_KEYS), lambda i: (0, 0, 0, 0)),
        ],
        out_specs=[
            pl.BlockSpec((TB, DIM), lambda i: (i, 0)),
            pl.BlockSpec((TB, HEADS * K), lambda i: (i, 0)),
            pl.BlockSpec((TB, HEADS * K), lambda i: (i, 0)),
        ],
        out_shape=[
            jax.ShapeDtypeStruct((nt, DIM), jnp.float32),
            jax.ShapeDtypeStruct((nt, HEADS * K), jnp.float32),
            jax.ShapeDtypeStruct((nt, HEADS * K), jnp.int32),
        ],
        interpret=interpret,
    )(x2d, gamma.reshape(1, DIM), Wq, Kmat)


NB = 4
TBB = N // NB


def kernel(x, gamma, Wq, keys_p, Wdown, Wup):
    x2d = x.reshape(N, DIM)
    Kmat = keys_p.transpose(2, 0, 3, 1)
    outs = []
    for i in range(NB):
        xn, w, idx = _phase_a(x2d[i * TBB:(i + 1) * TBB], gamma, Wq, Kmat)
        outs.append(_phase_b(xn, w, idx, Wdown, Wup))
    return jnp.concatenate(outs, axis=0).reshape(1, N, DIM)


NC, NS = 2, 16
NW = NC * NS
TPW = TBB // NW
KH = HEADS * K


def _gelu_w(hvec, wvec):
    z = hvec * 0.7071067811865476
    az = jnp.abs(z)
    t = 1.0 / (1.0 + 0.3275911 * az)
    poly = t * (0.254829592 + t * (-0.284496736 + t * (1.421413741
               + t * (-1.453152027 + t * 1.061405429))))
    erf_abs = 1.0 - poly * jnp.exp(-az * az)
    erf = jnp.where(z < 0, -erf_abs, erf_abs)
    return hvec * 0.5 * (1.0 + erf) * wvec


KHH = KH // 2


def _phase_b_body(xn_hbm, w_hbm, idx_hbm, dn_hbm, up_hbm, out_hbm,
                  idx_v, w_v, xn_v, dn_a, dn_b, up_a, up_b, out_v, g_smem,
                  sem_i, sem_da, sem_db, sem_ua, sem_ub, sem_x, sem_o):
    wid = lax.axis_index("s") * NC + lax.axis_index("c")
    base = wid * TPW

    pltpu.async_copy(idx_hbm.at[pl.ds(base, TPW)], idx_v, sem_i).wait()
    pltpu.async_copy(w_hbm.at[pl.ds(base, TPW)], w_v, sem_i).wait()

    lane = lax.iota(jnp.int32, 16)

    def idx_half(t, h):
        return idx_v.at[t, pl.ds(h * KHH, KHH)]

    pltpu.async_copy(dn_hbm.at[idx_half(0, 0)], dn_a, sem_da)
    pltpu.async_copy(dn_hbm.at[idx_half(0, 1)], dn_b, sem_db)
    pltpu.async_copy(up_hbm.at[idx_half(0, 0)], up_a, sem_ua)
    pltpu.async_copy(up_hbm.at[idx_half(0, 1)], up_b, sem_ub)
    pltpu.async_copy(xn_hbm.at[base], xn_v.at[0], sem_x)

    def h_half(rows, par):
        hs = []
        for kg in range(KHH // 8):
            def dslice(s, accs, _kg=kg, _par=par, _rows=rows):
                xs = xn_v[_par, pl.ds(s * 16, 16)]
                return tuple(
                    accs[j] + xs * _rows[_kg * 8 + j, pl.ds(s * 16, 16)]
                    for j in range(8))
            accs = lax.fori_loop(0, DIM // 16, dslice,
                                 tuple(jnp.zeros((16,), jnp.float32)
                                       for _ in range(8)))
            for j in range(8):
                hs.append(jnp.sum(accs[j]))
        return hs

    def token(t, carry):
        tok = base + t
        par = lax.rem(t, 2)
        pltpu.make_async_copy(xn_hbm.at[tok], xn_v.at[par], sem_x).wait()

        pltpu.make_async_copy(dn_hbm.at[idx_half(t, 0)], dn_a, sem_da).wait()
        h_scalars = h_half(dn_a, par)

        @pl.when(t < TPW - 1)
        def _():
            pltpu.async_copy(dn_hbm.at[idx_half(t + 1, 0)], dn_a, sem_da)

        pltpu.make_async_copy(dn_hbm.at[idx_half(t, 1)], dn_b, sem_db).wait()
        h_scalars += h_half(dn_b, par)

        @pl.when(t < TPW - 1)
        def _():
            pltpu.async_copy(dn_hbm.at[idx_half(t + 1, 1)], dn_b, sem_db)
            pltpu.async_copy(xn_hbm.at[tok + 1], xn_v.at[1 - par], sem_x)

        for kg in range(KH // 16):
            hv = jnp.zeros((16,), jnp.float32)
            for j in range(16):
                hv = jnp.where(lane == j, h_scalars[kg * 16 + j], hv)
            gv = _gelu_w(hv, w_v[t, pl.ds(kg * 16, 16)])
            for j in range(16):
                g_smem[kg * 16 + j] = jnp.sum(
                    jnp.where(lane == j, gv, 0.0))

        def combine(rows, koff, add):
            for chunk in range(3):
                def kstep(k, accs, _chunk=chunk, _koff=koff, _rows=rows):
                    b = jnp.full((16,), g_smem[_koff + k], jnp.float32)
                    return tuple(
                        accs[s] + b * _rows[k, pl.ds((_chunk * 16 + s) * 16, 16)]
                        for s in range(16))
                accs = lax.fori_loop(0, KHH, kstep,
                                     tuple(jnp.zeros((16,), jnp.float32)
                                           for _ in range(16)))
                for s in range(16):
                    sl = pl.ds((chunk * 16 + s) * 16, 16)
                    if add:
                        out_v[sl] = out_v[sl] + accs[s]
                    else:
                        out_v[sl] = accs[s]

        pltpu.make_async_copy(up_hbm.at[idx_half(t, 0)], up_a, sem_ua).wait()
        combine(up_a, 0, False)

        @pl.when(t < TPW - 1)
        def _():
            pltpu.async_copy(up_hbm.at[idx_half(t + 1, 0)], up_a, sem_ua)

        pltpu.make_async_copy(up_hbm.at[idx_half(t, 1)], up_b, sem_ub).wait()
        combine(up_b, KHH, True)

        @pl.when(t < TPW - 1)
        def _():
            pltpu.async_copy(up_hbm.at[idx_half(t + 1, 1)], up_b, sem_ub)

        pltpu.async_copy(out_v, out_hbm.at[tok], sem_o).wait()
        return carry

    lax.fori_loop(0, TPW, token, 0)


def _phase_b(xn, w, idx, Wdown, Wup):
    mesh = plsc.VectorSubcoreMesh(core_axis_name="c", subcore_axis_name="s",
                                  num_cores=NC, num_subcores=NS)
    f = pl.kernel(
        _phase_b_body,
        out_type=jax.ShapeDtypeStruct((TBB, DIM), jnp.float32),
        mesh=mesh,
        compiler_params=pltpu.CompilerParams(needs_layout_passes=False),
        scratch_types=[
            pltpu.VMEM((TPW, KH), jnp.int32),
            pltpu.VMEM((TPW, KH), jnp.float32),
            pltpu.VMEM((2, DIM), jnp.float32),
            pltpu.VMEM((KHH, DIM), jnp.float32),
            pltpu.VMEM((KHH, DIM), jnp.float32),
            pltpu.VMEM((KHH, DIM), jnp.float32),
            pltpu.VMEM((KHH, DIM), jnp.float32),
            pltpu.VMEM((DIM,), jnp.float32),
            pltpu.SMEM((KH,), jnp.float32),
            pltpu.SemaphoreType.DMA,
            pltpu.SemaphoreType.DMA,
            pltpu.SemaphoreType.DMA,
            pltpu.SemaphoreType.DMA,
            pltpu.SemaphoreType.DMA,
            pltpu.SemaphoreType.DMA,
            pltpu.SemaphoreType.DMA,
        ],
    )
    return f(xn, w, idx, Wdown, Wup)
